# baseline (speedup 1.0000x reference)
import functools
import math

import jax
import jax.numpy as jnp
from jax import lax
from jax.experimental import pallas as pl
from jax.experimental.pallas import tpu as pltpu
from jax.experimental.pallas import tpu_sc as plsc

D = 64
SCALE = math.sqrt(D)
NC, NS, L = 2, 16, 16
NW = NC * NS
CHUNK = 128


def _body(x_hbm, table_hbm, out_hbm, idx_v, rows_v, sem):
    n_chunks = x_hbm.shape[1]
    wid = lax.axis_index("s") * NC + lax.axis_index("c")
    pltpu.sync_copy(x_hbm.at[wid], idx_v)
    base = wid * (n_chunks * CHUNK)

    def chunk_step(c, _):
        pltpu.async_copy(table_hbm.at[idx_v.at[c]], rows_v, sem).wait()

        def row_step(r, _):
            for j in range(D // L):
                rows_v[r, pl.ds(j * L, L)] = rows_v[r, pl.ds(j * L, L)] * SCALE
            return 0

        lax.fori_loop(0, CHUNK, row_step, 0, unroll=4)
        pltpu.sync_copy(rows_v, out_hbm.at[pl.ds(base + c * CHUNK, CHUNK)])
        return 0

    lax.fori_loop(0, n_chunks, chunk_step, 0)


@jax.jit
def kernel(x, table):
    B, S = x.shape
    total = B * S
    n_chunks = total // (NW * CHUNK)
    x_flat = x.reshape(NW, n_chunks, CHUNK).astype(jnp.int32)

    run = pl.kernel(
        _body,
        out_type=jax.ShapeDtypeStruct((total, D), jnp.float32),
        mesh=plsc.VectorSubcoreMesh(
            core_axis_name="c", subcore_axis_name="s", num_cores=NC,
            num_subcores=NS),
        scratch_types=[
            pltpu.VMEM((n_chunks, CHUNK), jnp.int32),
            pltpu.VMEM((CHUNK, D), jnp.float32),
            pltpu.SemaphoreType.DMA,
        ],
        compiler_params=pltpu.CompilerParams(use_tc_tiling_on_sc=False),
    )
    out = run(x_flat, table)
    return out.reshape(B, S, D)


# speedup vs baseline: 1.1643x; 1.1643x over previous
import math

import jax
import jax.numpy as jnp
from jax import lax
from jax.experimental import pallas as pl
from jax.experimental.pallas import tpu as pltpu
from jax.experimental.pallas import tpu_sc as plsc

D = 64
SCALE = math.sqrt(D)
NC, NS, L = 2, 16, 16
NW = NC * NS
CHUNK = 128
NBUF = 5


def _body(x_hbm, table_hbm, out_hbm, idx_v, rows_v, gsems, osems):
    n_chunks = x_hbm.shape[1]
    wid = lax.axis_index("s") * NC + lax.axis_index("c")
    pltpu.sync_copy(x_hbm.at[wid], idx_v)
    base = wid * (n_chunks * CHUNK)

    def gather(c, b):
        return pltpu.make_async_copy(
            table_hbm.at[idx_v.at[c]], rows_v.at[b], gsems[b])

    def ocopy(c, b):
        return pltpu.make_async_copy(
            rows_v.at[b], out_hbm.at[pl.ds(base + c * CHUNK, CHUNK)],
            osems[b])

    def scale(b):
        def row_step(r, _):
            for j in range(D // L):
                rows_v[b, r, pl.ds(j * L, L)] = (
                    rows_v[b, r, pl.ds(j * L, L)] * SCALE)
            return 0

        lax.fori_loop(0, CHUNK, row_step, 0, unroll=4)

    for b in range(NBUF - 1):
        gather(b, b).start()

    def step(c, b):
        prev_b = (b + NBUF - 1) % NBUF
        @pl.when(c >= 1)
        def _():
            ocopy(c - 1, prev_b).wait()

        @pl.when(c + NBUF - 1 < n_chunks)
        def _():
            gather(c + NBUF - 1, prev_b).start()

        gather(c, b).wait()
        scale(b)
        ocopy(c, b).start()

    def outer(k, _):
        for b in range(NBUF):
            step(k * NBUF + b, b)
        return 0

    lax.fori_loop(0, n_chunks // NBUF, outer, 0)
    ocopy(n_chunks - 1, (n_chunks - 1) % NBUF).wait()


@jax.jit
def kernel(x, table):
    B, S = x.shape
    total = B * S
    n_chunks = total // (NW * CHUNK)
    x_flat = x.reshape(NW, n_chunks, CHUNK).astype(jnp.int32)

    run = pl.kernel(
        _body,
        out_type=jax.ShapeDtypeStruct((total, D), jnp.float32),
        mesh=plsc.VectorSubcoreMesh(
            core_axis_name="c", subcore_axis_name="s", num_cores=NC,
            num_subcores=NS),
        scratch_types=[
            pltpu.VMEM((n_chunks, CHUNK), jnp.int32),
            pltpu.VMEM((NBUF, CHUNK, D), jnp.float32),
            [pltpu.SemaphoreType.DMA] * NBUF,
            [pltpu.SemaphoreType.DMA] * NBUF,
        ],
        compiler_params=pltpu.CompilerParams(use_tc_tiling_on_sc=False),
    )
    out = run(x_flat, table)
    return out.reshape(B, S, D)
